# baseline (speedup 1.0000x reference)
import functools

import jax
import jax.numpy as jnp
from jax import lax
from jax.experimental import pallas as pl
from jax.experimental.pallas import tpu as pltpu
from jax.experimental.pallas import tpu_sc as plsc

N_L = 4096
N_M = 16384
F1 = 64
F2 = 64
H = 128
TQ = 256
GRID = N_M // TQ
K = 3
EPS = 1e-5

NC = 2
NS = 16
NW = NC * NS
PER_W = N_M * K // NW
CHUNK = 128
NCHUNK = PER_W // CHUNK

_HIGHEST = jax.lax.Precision.HIGHEST
_DEF = jax.lax.Precision.DEFAULT


def _dot(a, b, precision=_DEF):
    return jax.lax.dot_general(a, b, (((1,), (0,)), ((), ())),
                               precision=precision,
                               preferred_element_type=jnp.float32)


def _k1_body(q_ref, pT_ref, idx_ref, w_ref):
    q = q_ref[...]
    pT = pT_ref[...]
    qn = jnp.sum(q * q, axis=1, keepdims=True)
    pn = jnp.sum(pT * pT, axis=0, keepdims=True)
    g = _dot(q, pT, _HIGHEST)
    d2 = qn + pn - 2.0 * g

    iota = jax.lax.broadcasted_iota(jnp.int32, (TQ, N_L), 1)
    idxs, ws = [], []
    wsum = jnp.zeros((TQ, 1), jnp.float32)
    for k in range(K):
        m = jnp.min(d2, axis=1, keepdims=True)
        idx = jnp.min(jnp.where(d2 == m, iota, N_L), axis=1,
                      keepdims=True)
        wk = 1.0 / jnp.maximum(m, 1e-12)
        idxs.append(idx)
        ws.append(wk)
        wsum = wsum + wk
        if k < K - 1:
            d2 = jnp.where(iota == idx, jnp.inf, d2)
    inv = 1.0 / wsum
    idx_ref[...] = jnp.concatenate(idxs, axis=1)
    w_ref[...] = jnp.concatenate([w * inv for w in ws], axis=1)


def _sc_gather_body(pf_hbm, idx_hbm, out_hbm, idx_v, rows_v, sem):
    wid = lax.axis_index("s") * NC + lax.axis_index("c")
    base = wid * PER_W
    for c in range(NCHUNK):
        off = base + c * CHUNK
        pltpu.sync_copy(idx_hbm.at[pl.ds(off, CHUNK)], idx_v)
        pltpu.async_copy(pf_hbm.at[idx_v], rows_v, sem).wait()
        pltpu.sync_copy(rows_v, out_hbm.at[pl.ds(off, CHUNK)])


def _k2_body(q_ref, of_ref, g_ref, w_ref, w0c_ref, w0f_ref, w0i_ref,
             b0_ref, y0_ref, s_ref, ss_ref):
    w = w_ref[...]
    g = g_ref[..., :F1]
    interp = jnp.sum(g * w[:, :, None], axis=1)
    y0 = (_dot(q_ref[...], w0c_ref[...]) + _dot(of_ref[...], w0f_ref[...])
          + _dot(interp, w0i_ref[...]) + b0_ref[...])
    y0_ref[...] = y0
    s_ref[...] = jnp.sum(y0, axis=0, keepdims=True)[None]
    ss_ref[...] = jnp.sum(y0 * y0, axis=0, keepdims=True)[None]


def _k3_body(y0_ref, sc_ref, sh_ref, w1_ref, b1_ref, y1_ref, s_ref, ss_ref):
    h = jnp.maximum(y0_ref[...] * sc_ref[...] + sh_ref[...], 0.0)
    y1 = _dot(h, w1_ref[...]) + b1_ref[...]
    y1_ref[...] = y1
    s_ref[...] = jnp.sum(y1, axis=0, keepdims=True)[None]
    ss_ref[...] = jnp.sum(y1 * y1, axis=0, keepdims=True)[None]


def _k4_body(y1_ref, sc_ref, sh_ref, o_ref):
    o_ref[...] = jnp.maximum(y1_ref[...] * sc_ref[...] + sh_ref[...], 0.0)


def _bn_coeffs(s, ss, g, be):
    mu = jnp.sum(s, axis=0)[0] / N_M
    var = jnp.sum(ss, axis=0)[0] / N_M - mu * mu
    scale = g / jnp.sqrt(var + EPS)
    shift = be - mu * scale
    return scale[None, :], shift[None, :]


@jax.jit
def kernel(prop_coords, prop_feats, orig_coords, orig_feats,
           W0, b0, g0, be0, W1, b1, g1, be1):
    qpad = jnp.pad(orig_coords, ((0, 0), (0, 5)))
    pT = jnp.pad(prop_coords, ((0, 0), (0, 5))).T
    w0c = jnp.pad(W0[:3], ((0, 5), (0, 0)))
    w0f = W0[3:3 + F2]
    w0i = W0[3 + F2:]

    full = lambda shp: pl.BlockSpec(shp, lambda i: (0,) * len(shp))
    row = lambda w: pl.BlockSpec((TQ, w), lambda i: (i, 0))
    stat = pl.BlockSpec((1, 1, H), lambda i: (i, 0, 0))
    statshape = jax.ShapeDtypeStruct((GRID, 1, H), jnp.float32)

    idxq, w3 = pl.pallas_call(
        _k1_body,
        grid=(GRID,),
        in_specs=[row(8), full((8, N_L))],
        out_specs=[row(K), row(K)],
        out_shape=[jax.ShapeDtypeStruct((N_M, K), jnp.int32),
                   jax.ShapeDtypeStruct((N_M, K), jnp.float32)],
    )(qpad, pT)

    sc_gather = pl.kernel(
        _sc_gather_body,
        out_type=jax.ShapeDtypeStruct((N_M * K, 2 * F1), jnp.float32),
        mesh=plsc.VectorSubcoreMesh(core_axis_name="c", subcore_axis_name="s"),
        scratch_types=[pltpu.VMEM((CHUNK,), jnp.int32),
                       pltpu.VMEM((CHUNK, 2 * F1), jnp.float32),
                       pltpu.SemaphoreType.DMA],
    )
    pf_pad = jnp.pad(prop_feats, ((0, 0), (0, 64)))
    gath = sc_gather(pf_pad, idxq.reshape(N_M * K))
    gath = gath.reshape(N_M, K, 2 * F1)

    y0, s0, ss0 = pl.pallas_call(
        _k2_body,
        grid=(GRID,),
        in_specs=[row(8), row(F2),
                  pl.BlockSpec((TQ, K, 2 * F1), lambda i: (i, 0, 0)), row(K),
                  full((8, H)), full((F2, H)), full((F1, H)), full((1, H))],
        out_specs=[row(H), stat, stat],
        out_shape=[jax.ShapeDtypeStruct((N_M, H), jnp.float32),
                   statshape, statshape],
    )(qpad, orig_feats, gath, w3, w0c, w0f, w0i, b0[None, :])

    sc0, sh0 = _bn_coeffs(s0, ss0, g0, be0)
    y1, s1, ss1 = pl.pallas_call(
        _k3_body,
        grid=(GRID,),
        in_specs=[row(H), full((1, H)), full((1, H)), full((H, H)),
                  full((1, H))],
        out_specs=[row(H), stat, stat],
        out_shape=[jax.ShapeDtypeStruct((N_M, H), jnp.float32),
                   statshape, statshape],
    )(y0, sc0, sh0, W1, b1[None, :])

    sc1, sh1 = _bn_coeffs(s1, ss1, g1, be1)
    out = pl.pallas_call(
        _k4_body,
        grid=(GRID,),
        in_specs=[row(H), full((1, H)), full((1, H))],
        out_specs=row(H),
        out_shape=jax.ShapeDtypeStruct((N_M, H), jnp.float32),
    )(y1, sc1, sh1)
    return out


# speedup vs baseline: 1.3478x; 1.3478x over previous
import functools

import jax
import jax.numpy as jnp
from jax import lax
from jax.experimental import pallas as pl
from jax.experimental.pallas import tpu as pltpu
from jax.experimental.pallas import tpu_sc as plsc

N_L = 4096
N_M = 16384
F1 = 64
F2 = 64
H = 128
TQ = 512
GRID = N_M // TQ
K = 3
EPS = 1e-5

NC = 2
NS = 16
NW = NC * NS
PER_W = N_M * K // NW
CHUNK = 128
NCHUNK = PER_W // CHUNK

_HIGHEST = jax.lax.Precision.HIGHEST
_DEF = jax.lax.Precision.DEFAULT


def _dot(a, b, precision=_DEF):
    return jax.lax.dot_general(a, b, (((1,), (0,)), ((), ())),
                               precision=precision,
                               preferred_element_type=jnp.float32)


NSLAB = 32
SLABW = N_L // NSLAB


def _k1_body(q_ref, pT_ref, slab_ref, idx_ref, w_ref):
    q = q_ref[...]
    pT = pT_ref[...]
    qn = jnp.sum(q * q, axis=1, keepdims=True)
    pn = jnp.sum(pT * pT, axis=0, keepdims=True)
    g = _dot(q, pT, _HIGHEST)
    d2 = jnp.maximum(qn + pn - 2.0 * g, 0.0)

    b = jax.lax.bitcast_convert_type(d2, jnp.int32)
    keys = jax.lax.bitcast_convert_type((b & ~(NSLAB - 1)) | slab_ref[...],
                                        jnp.float32)

    lane_iota = jax.lax.broadcasted_iota(jnp.int32, (TQ, SLABW), 1)
    idxs, ws = [], []
    wsum = jnp.zeros((TQ, 1), jnp.float32)
    for k in range(K):
        cmin = keys[:, :SLABW]
        for s in range(1, NSLAB):
            cmin = jnp.minimum(cmin, keys[:, s * SLABW:(s + 1) * SLABW])
        m = jnp.min(cmin, axis=1, keepdims=True)
        lane = jnp.min(jnp.where(cmin == m, lane_iota, SLABW),
                       axis=1, keepdims=True)
        mi = jax.lax.bitcast_convert_type(m, jnp.int32)
        slab = mi & (NSLAB - 1)
        d2t = jax.lax.bitcast_convert_type(mi & ~(NSLAB - 1), jnp.float32)
        idxs.append(slab * SLABW + lane)
        wk = 1.0 / jnp.maximum(d2t, 1e-12)
        ws.append(wk)
        wsum = wsum + wk
        if k < K - 1:
            keys = jnp.where(keys == m, jnp.inf, keys)
    inv = 1.0 / wsum
    idx_ref[...] = jnp.concatenate(idxs, axis=1)
    w_ref[...] = jnp.concatenate([w * inv for w in ws], axis=1)


def _sc_gather_body(pf_hbm, idx_hbm, out_hbm, idx_v, rows_v, sem):
    wid = lax.axis_index("s") * NC + lax.axis_index("c")
    base = wid * PER_W
    for c in range(NCHUNK):
        off = base + c * CHUNK
        pltpu.sync_copy(idx_hbm.at[pl.ds(off, CHUNK)], idx_v)
        pltpu.async_copy(pf_hbm.at[idx_v], rows_v, sem).wait()
        pltpu.sync_copy(rows_v, out_hbm.at[pl.ds(off, CHUNK)])


def _k2_body(q_ref, of_ref, g_ref, w_ref, w0c_ref, w0f_ref, w0i_ref,
             b0_ref, y0_ref, s_ref, ss_ref):
    w = w_ref[...]
    g = g_ref[..., :F1]
    interp = jnp.sum(g * w[:, :, None], axis=1)
    y0 = (_dot(q_ref[...], w0c_ref[...]) + _dot(of_ref[...], w0f_ref[...])
          + _dot(interp, w0i_ref[...]) + b0_ref[...])
    y0_ref[...] = y0
    s_ref[...] = jnp.sum(y0, axis=0, keepdims=True)[None]
    ss_ref[...] = jnp.sum(y0 * y0, axis=0, keepdims=True)[None]


def _k3_body(y0_ref, sc_ref, sh_ref, w1_ref, b1_ref, y1_ref, s_ref, ss_ref):
    h = jnp.maximum(y0_ref[...] * sc_ref[...] + sh_ref[...], 0.0)
    y1 = _dot(h, w1_ref[...]) + b1_ref[...]
    y1_ref[...] = y1
    s_ref[...] = jnp.sum(y1, axis=0, keepdims=True)[None]
    ss_ref[...] = jnp.sum(y1 * y1, axis=0, keepdims=True)[None]


def _k4_body(y1_ref, sc_ref, sh_ref, o_ref):
    o_ref[...] = jnp.maximum(y1_ref[...] * sc_ref[...] + sh_ref[...], 0.0)


def _bn_coeffs(s, ss, g, be):
    mu = jnp.sum(s, axis=0)[0] / N_M
    var = jnp.sum(ss, axis=0)[0] / N_M - mu * mu
    scale = g / jnp.sqrt(var + EPS)
    shift = be - mu * scale
    return scale[None, :], shift[None, :]


@jax.jit
def kernel(prop_coords, prop_feats, orig_coords, orig_feats,
           W0, b0, g0, be0, W1, b1, g1, be1):
    qpad = jnp.pad(orig_coords, ((0, 0), (0, 5)))
    pT = jnp.pad(prop_coords, ((0, 0), (0, 5))).T
    w0c = jnp.pad(W0[:3], ((0, 5), (0, 0)))
    w0f = W0[3:3 + F2]
    w0i = W0[3 + F2:]

    def _split3(x):
        h = x.astype(jnp.bfloat16).astype(jnp.float32)
        m = (x - h).astype(jnp.bfloat16).astype(jnp.float32)
        l = (x - h - m).astype(jnp.bfloat16).astype(jnp.float32)
        return h, m, l

    qh, qm, ql = _split3(-2.0 * orig_coords)
    qn = jnp.sum(orig_coords * orig_coords, axis=1, keepdims=True)
    qnh, qnm, qnl = _split3(qn)
    one_q = jnp.ones((N_M, 1), jnp.float32)
    qcat = jnp.pad(jnp.concatenate(
        [qh, qh, qh, qm, qm, ql, qnh, qnm, qnl, one_q, one_q, one_q],
        axis=1), ((0, 0), (0, 8))).astype(jnp.bfloat16)

    ph, pm_, pl_ = _split3(prop_coords)
    pn = jnp.sum(prop_coords * prop_coords, axis=1, keepdims=True)
    pnh, pnm, pnl = _split3(pn)
    one_p = jnp.ones((N_L, 1), jnp.float32)
    pcatT = jnp.pad(jnp.concatenate(
        [ph, pm_, pl_, ph, pm_, ph, one_p, one_p, one_p, pnh, pnm, pnl],
        axis=1), ((0, 0), (0, 8))).astype(jnp.bfloat16).T

    slab_row = (jnp.arange(N_L, dtype=jnp.int32) // SLABW)[None, :]
    full = lambda shp: pl.BlockSpec(shp, lambda i: (0,) * len(shp))
    row = lambda w: pl.BlockSpec((TQ, w), lambda i: (i, 0))
    stat = pl.BlockSpec((1, 1, H), lambda i: (i, 0, 0))
    statshape = jax.ShapeDtypeStruct((GRID, 1, H), jnp.float32)

    idxq, w3 = pl.pallas_call(
        _k1_body,
        grid=(GRID,),
        in_specs=[row(8), full((8, N_L)), full((1, N_L))],
        out_specs=[row(K), row(K)],
        out_shape=[jax.ShapeDtypeStruct((N_M, K), jnp.int32),
                   jax.ShapeDtypeStruct((N_M, K), jnp.float32)],
    )(qpad, pT, slab_row)

    sc_gather = pl.kernel(
        _sc_gather_body,
        out_type=jax.ShapeDtypeStruct((N_M * K, 2 * F1), jnp.float32),
        mesh=plsc.VectorSubcoreMesh(core_axis_name="c", subcore_axis_name="s"),
        scratch_types=[pltpu.VMEM((CHUNK,), jnp.int32),
                       pltpu.VMEM((CHUNK, 2 * F1), jnp.float32),
                       pltpu.SemaphoreType.DMA],
    )
    pf_pad = jnp.pad(prop_feats, ((0, 0), (0, 64)))
    gath = sc_gather(pf_pad, idxq.reshape(N_M * K))
    gath = gath.reshape(N_M, K, 2 * F1)

    y0, s0, ss0 = pl.pallas_call(
        _k2_body,
        grid=(GRID,),
        in_specs=[row(8), row(F2),
                  pl.BlockSpec((TQ, K, 2 * F1), lambda i: (i, 0, 0)), row(K),
                  full((8, H)), full((F2, H)), full((F1, H)), full((1, H))],
        out_specs=[row(H), stat, stat],
        out_shape=[jax.ShapeDtypeStruct((N_M, H), jnp.float32),
                   statshape, statshape],
    )(qpad, orig_feats, gath, w3, w0c, w0f, w0i, b0[None, :])

    sc0, sh0 = _bn_coeffs(s0, ss0, g0, be0)
    y1, s1, ss1 = pl.pallas_call(
        _k3_body,
        grid=(GRID,),
        in_specs=[row(H), full((1, H)), full((1, H)), full((H, H)),
                  full((1, H))],
        out_specs=[row(H), stat, stat],
        out_shape=[jax.ShapeDtypeStruct((N_M, H), jnp.float32),
                   statshape, statshape],
    )(y0, sc0, sh0, W1, b1[None, :])

    sc1, sh1 = _bn_coeffs(s1, ss1, g1, be1)
    out = pl.pallas_call(
        _k4_body,
        grid=(GRID,),
        in_specs=[row(H), full((1, H)), full((1, H))],
        out_specs=row(H),
        out_shape=jax.ShapeDtypeStruct((N_M, H), jnp.float32),
    )(y1, sc1, sh1)
    return out


# speedup vs baseline: 1.4983x; 1.1117x over previous
import jax
import jax.numpy as jnp
from jax import lax
from jax.experimental import pallas as pl
from jax.experimental.pallas import tpu as pltpu
from jax.experimental.pallas import tpu_sc as plsc

N_L = 4096
N_M = 16384
F1 = 64
F2 = 64
H = 128
TQ = 512
GRID = N_M // TQ
K = 3
EPS = 1e-5

NC = 2
NS = 16
NW = NC * NS
QPER_W = N_M // NW
CHUNK = 128
NCHUNK = QPER_W // CHUNK

_HIGHEST = jax.lax.Precision.HIGHEST
_DEF = jax.lax.Precision.DEFAULT

NSLAB = 32
SLABW = N_L // NSLAB


def _dot(a, b, precision=_DEF):
    return jax.lax.dot_general(a, b, (((1,), (0,)), ((), ())),
                               precision=precision,
                               preferred_element_type=jnp.float32)


def _k1_body(q_ref, pT_ref, slab_ref, idx_ref, w_ref):
    q = q_ref[...]
    pT = pT_ref[...]
    qn = jnp.sum(q * q, axis=1, keepdims=True)
    pn = jnp.sum(pT * pT, axis=0, keepdims=True)
    g = _dot(q, pT, _HIGHEST)
    d2 = jnp.maximum(qn + pn - 2.0 * g, 0.0)

    b = jax.lax.bitcast_convert_type(d2, jnp.int32)
    keys = jax.lax.bitcast_convert_type((b & ~(NSLAB - 1)) | slab_ref[...],
                                        jnp.float32)

    lane_iota = jax.lax.broadcasted_iota(jnp.int32, (TQ, SLABW), 1)
    idxs, ws = [], []
    wsum = jnp.zeros((TQ, 1), jnp.float32)
    for k in range(K):
        cmin = keys[:, :SLABW]
        for s in range(1, NSLAB):
            cmin = jnp.minimum(cmin, keys[:, s * SLABW:(s + 1) * SLABW])
        m = jnp.min(cmin, axis=1, keepdims=True)
        lane = jnp.min(jnp.where(cmin == m, lane_iota, SLABW),
                       axis=1, keepdims=True)
        mi = jax.lax.bitcast_convert_type(m, jnp.int32)
        slab = mi & (NSLAB - 1)
        d2t = jax.lax.bitcast_convert_type(mi & ~(NSLAB - 1), jnp.float32)
        idxs.append(slab * SLABW + lane)
        wk = 1.0 / jnp.maximum(d2t, 1e-12)
        ws.append(wk)
        wsum = wsum + wk
        if k < K - 1:
            keys = jnp.where(keys == m, jnp.inf, keys)
    inv = 1.0 / wsum
    idx_ref[...] = jnp.concatenate(idxs, axis=1)
    w_ref[...] = jnp.concatenate([w * inv for w in ws], axis=1)


def _sc_gather_body(pf_hbm, idx_hbm, out_hbm, idx_v, rows_v, sem):
    wid = lax.axis_index("s") * NC + lax.axis_index("c")
    qbase = wid * QPER_W
    for k in range(K):
        for c in range(NCHUNK):
            off = k * N_M + qbase + c * CHUNK
            pltpu.sync_copy(idx_hbm.at[pl.ds(off, CHUNK)], idx_v)
            pltpu.async_copy(pf_hbm.at[idx_v], rows_v, sem).wait()
            pltpu.sync_copy(rows_v, out_hbm.at[pl.ds(off, CHUNK)])


def _k2_body(q_ref, of_ref, g0_ref, g1_ref, g2_ref, w_ref, w0c_ref,
             w0f_ref, w0i_ref, b0_ref, y0_ref, s_ref, ss_ref):
    w = w_ref[...]
    interp = (g0_ref[:, :F1] * w[:, 0:1] + g1_ref[:, :F1] * w[:, 1:2]
              + g2_ref[:, :F1] * w[:, 2:3])
    y0 = (_dot(q_ref[...], w0c_ref[...]) + _dot(of_ref[...], w0f_ref[...])
          + _dot(interp, w0i_ref[...]) + b0_ref[...])
    y0_ref[...] = y0
    s_ref[...] = jnp.sum(y0, axis=0, keepdims=True)[None]
    ss_ref[...] = jnp.sum(y0 * y0, axis=0, keepdims=True)[None]


def _bn_scale_shift(s_ref, ss_ref, g_ref, be_ref):
    mu = jnp.sum(s_ref[:, 0, :], axis=0, keepdims=True) * (1.0 / N_M)
    var = jnp.sum(ss_ref[:, 0, :], axis=0, keepdims=True) * (1.0 / N_M) - mu * mu
    scale = g_ref[...] / jnp.sqrt(var + EPS)
    shift = be_ref[...] - mu * scale
    return scale, shift


def _k3_body(y0_ref, s0_ref, ss0_ref, g0_ref, be0_ref, w1_ref, b1_ref,
             y1_ref, s_ref, ss_ref):
    scale, shift = _bn_scale_shift(s0_ref, ss0_ref, g0_ref, be0_ref)
    h = jnp.maximum(y0_ref[...] * scale + shift, 0.0)
    y1 = _dot(h, w1_ref[...]) + b1_ref[...]
    y1_ref[...] = y1
    s_ref[...] = jnp.sum(y1, axis=0, keepdims=True)[None]
    ss_ref[...] = jnp.sum(y1 * y1, axis=0, keepdims=True)[None]


def _k4_body(y1_ref, s1_ref, ss1_ref, g1_ref, be1_ref, o_ref):
    scale, shift = _bn_scale_shift(s1_ref, ss1_ref, g1_ref, be1_ref)
    o_ref[...] = jnp.maximum(y1_ref[...] * scale + shift, 0.0)


@jax.jit
def kernel(prop_coords, prop_feats, orig_coords, orig_feats,
           W0, b0, g0, be0, W1, b1, g1, be1):
    qpad = jnp.pad(orig_coords, ((0, 0), (0, 5)))
    pT = jnp.pad(prop_coords, ((0, 0), (0, 5))).T
    w0c = jnp.pad(W0[:3], ((0, 5), (0, 0)))
    w0f = W0[3:3 + F2]
    w0i = W0[3 + F2:]

    slab_row = (jnp.arange(N_L, dtype=jnp.int32) // SLABW)[None, :]
    full = lambda shp: pl.BlockSpec(shp, lambda i: (0,) * len(shp))
    row = lambda w: pl.BlockSpec((TQ, w), lambda i: (i, 0))
    stat = pl.BlockSpec((1, 1, H), lambda i: (i, 0, 0))
    statshape = jax.ShapeDtypeStruct((GRID, 1, H), jnp.float32)

    idxq, w3 = pl.pallas_call(
        _k1_body,
        grid=(GRID,),
        in_specs=[row(8), full((8, N_L)), full((1, N_L))],
        out_specs=[row(K), row(K)],
        out_shape=[jax.ShapeDtypeStruct((N_M, K), jnp.int32),
                   jax.ShapeDtypeStruct((N_M, K), jnp.float32)],
    )(qpad, pT, slab_row)

    sc_gather = pl.kernel(
        _sc_gather_body,
        out_type=jax.ShapeDtypeStruct((K * N_M, 2 * F1), jnp.float32),
        mesh=plsc.VectorSubcoreMesh(core_axis_name="c", subcore_axis_name="s"),
        scratch_types=[pltpu.VMEM((CHUNK,), jnp.int32),
                       pltpu.VMEM((CHUNK, 2 * F1), jnp.float32),
                       pltpu.SemaphoreType.DMA],
    )
    pf_pad = jnp.pad(prop_feats, ((0, 0), (0, 64)))
    gath = sc_gather(pf_pad, idxq.T.reshape(K * N_M))

    gblk = lambda k: pl.BlockSpec((TQ, 2 * F1), lambda i, k=k: (k * GRID + i, 0))
    y0, s0, ss0 = pl.pallas_call(
        _k2_body,
        grid=(GRID,),
        in_specs=[row(8), row(F2), gblk(0), gblk(1), gblk(2), row(K),
                  full((8, H)), full((F2, H)), full((F1, H)), full((1, H))],
        out_specs=[row(H), stat, stat],
        out_shape=[jax.ShapeDtypeStruct((N_M, H), jnp.float32),
                   statshape, statshape],
    )(qpad, orig_feats, gath, gath, gath, w3, w0c, w0f, w0i, b0[None, :])

    y1, s1, ss1 = pl.pallas_call(
        _k3_body,
        grid=(GRID,),
        in_specs=[row(H), full((GRID, 1, H)), full((GRID, 1, H)),
                  full((1, H)), full((1, H)), full((H, H)), full((1, H))],
        out_specs=[row(H), stat, stat],
        out_shape=[jax.ShapeDtypeStruct((N_M, H), jnp.float32),
                   statshape, statshape],
    )(y0, s0, ss0, g0[None, :], be0[None, :], W1, b1[None, :])

    out = pl.pallas_call(
        _k4_body,
        grid=(GRID,),
        in_specs=[row(H), full((GRID, 1, H)), full((GRID, 1, H)),
                  full((1, H)), full((1, H))],
        out_specs=row(H),
        out_shape=jax.ShapeDtypeStruct((N_M, H), jnp.float32),
    )(y1, s1, ss1, g1[None, :], be1[None, :])
    return out


# speedup vs baseline: 1.5300x; 1.0212x over previous
import jax
import jax.numpy as jnp
from jax import lax
from jax.experimental import pallas as pl
from jax.experimental.pallas import tpu as pltpu
from jax.experimental.pallas import tpu_sc as plsc

N_L = 4096
N_M = 16384
F1 = 64
F2 = 64
H = 128
TQ = 512
GRID = N_M // TQ
K = 3
EPS = 1e-5

NC = 2
NS = 16
NW = NC * NS
QPER_W = N_M // NW
CHUNK = 128
NCHUNK = QPER_W // CHUNK

_HIGHEST = jax.lax.Precision.HIGHEST
_DEF = jax.lax.Precision.DEFAULT

NSLAB = 32
SLABW = N_L // NSLAB


def _dot(a, b, precision=_DEF):
    return jax.lax.dot_general(a, b, (((1,), (0,)), ((), ())),
                               precision=precision,
                               preferred_element_type=jnp.float32)


def _k1_body(q_ref, pT_ref, slab_ref, idx_ref, w_ref):
    q = q_ref[...]
    pT = pT_ref[...]
    qn = jnp.sum(q * q, axis=1, keepdims=True)
    pn = jnp.sum(pT * pT, axis=0, keepdims=True)
    g = _dot(q, pT, _HIGHEST)
    d2 = jnp.maximum(qn + pn - 2.0 * g, 0.0)

    b = jax.lax.bitcast_convert_type(d2, jnp.int32)
    keys = jax.lax.bitcast_convert_type((b & ~(NSLAB - 1)) | slab_ref[...],
                                        jnp.float32)

    lane_iota = jax.lax.broadcasted_iota(jnp.int32, (TQ, SLABW), 1)
    idxs, ws = [], []
    wsum = jnp.zeros((TQ, 1), jnp.float32)
    for k in range(K):
        cmin = keys[:, :SLABW]
        for s in range(1, NSLAB):
            cmin = jnp.minimum(cmin, keys[:, s * SLABW:(s + 1) * SLABW])
        m = jnp.min(cmin, axis=1, keepdims=True)
        lane = jnp.min(jnp.where(cmin == m, lane_iota, SLABW),
                       axis=1, keepdims=True)
        mi = jax.lax.bitcast_convert_type(m, jnp.int32)
        slab = mi & (NSLAB - 1)
        d2t = jax.lax.bitcast_convert_type(mi & ~(NSLAB - 1), jnp.float32)
        idxs.append(slab * SLABW + lane)
        wk = 1.0 / jnp.maximum(d2t, 1e-12)
        ws.append(wk)
        wsum = wsum + wk
        if k < K - 1:
            keys = jnp.where(keys == m, jnp.inf, keys)
    inv = 1.0 / wsum
    idx_ref[...] = jnp.concatenate(idxs, axis=1)
    w_ref[...] = jnp.concatenate([w * inv for w in ws], axis=1)


def _sc_gather_body(pf_hbm, idx_hbm, out_hbm, idx_v, rows_v, sem):
    wid = lax.axis_index("s") * NC + lax.axis_index("c")
    qbase = wid * QPER_W
    for k in range(K):
        pltpu.sync_copy(idx_hbm.at[pl.ds(k * N_M + qbase, QPER_W)], idx_v)
        copies = []
        for c in range(NCHUNK):
            copies.append(pltpu.async_copy(
                pf_hbm.at[idx_v.at[pl.ds(c * CHUNK, CHUNK)]],
                rows_v.at[pl.ds(c * CHUNK, CHUNK)], sem))
        for cp in copies:
            cp.wait()
        pltpu.sync_copy(rows_v, out_hbm.at[pl.ds(k * N_M + qbase, QPER_W)])


def _k2_body(q_ref, of_ref, g0_ref, g1_ref, g2_ref, w_ref, w0c_ref,
             w0f_ref, w0i_ref, b0_ref, y0_ref, s_ref, ss_ref):
    w = w_ref[...]
    interp = (g0_ref[:, :F1] * w[:, 0:1] + g1_ref[:, :F1] * w[:, 1:2]
              + g2_ref[:, :F1] * w[:, 2:3])
    y0 = (_dot(q_ref[...], w0c_ref[...]) + _dot(of_ref[...], w0f_ref[...])
          + _dot(interp, w0i_ref[...]) + b0_ref[...])
    y0_ref[...] = y0
    s_ref[...] = jnp.sum(y0, axis=0, keepdims=True)[None]
    ss_ref[...] = jnp.sum(y0 * y0, axis=0, keepdims=True)[None]


def _bn_scale_shift(s_ref, ss_ref, g_ref, be_ref):
    mu = jnp.sum(s_ref[:, 0, :], axis=0, keepdims=True) * (1.0 / N_M)
    var = jnp.sum(ss_ref[:, 0, :], axis=0, keepdims=True) * (1.0 / N_M) - mu * mu
    scale = g_ref[...] / jnp.sqrt(var + EPS)
    shift = be_ref[...] - mu * scale
    return scale, shift


def _k3_body(y0_ref, s0_ref, ss0_ref, g0_ref, be0_ref, w1_ref, b1_ref,
             y1_ref, s_ref, ss_ref):
    scale, shift = _bn_scale_shift(s0_ref, ss0_ref, g0_ref, be0_ref)
    h = jnp.maximum(y0_ref[...] * scale + shift, 0.0)
    y1 = _dot(h, w1_ref[...]) + b1_ref[...]
    y1_ref[...] = y1
    s_ref[...] = jnp.sum(y1, axis=0, keepdims=True)[None]
    ss_ref[...] = jnp.sum(y1 * y1, axis=0, keepdims=True)[None]


def _k4_body(y1_ref, s1_ref, ss1_ref, g1_ref, be1_ref, o_ref):
    scale, shift = _bn_scale_shift(s1_ref, ss1_ref, g1_ref, be1_ref)
    o_ref[...] = jnp.maximum(y1_ref[...] * scale + shift, 0.0)


@jax.jit
def kernel(prop_coords, prop_feats, orig_coords, orig_feats,
           W0, b0, g0, be0, W1, b1, g1, be1):
    qpad = jnp.pad(orig_coords, ((0, 0), (0, 5)))
    pT = jnp.pad(prop_coords, ((0, 0), (0, 5))).T
    w0c = jnp.pad(W0[:3], ((0, 5), (0, 0)))
    w0f = W0[3:3 + F2]
    w0i = W0[3 + F2:]

    slab_row = (jnp.arange(N_L, dtype=jnp.int32) // SLABW)[None, :]
    full = lambda shp: pl.BlockSpec(shp, lambda i: (0,) * len(shp))
    row = lambda w: pl.BlockSpec((TQ, w), lambda i: (i, 0))
    stat = pl.BlockSpec((1, 1, H), lambda i: (i, 0, 0))
    statshape = jax.ShapeDtypeStruct((GRID, 1, H), jnp.float32)

    idxq, w3 = pl.pallas_call(
        _k1_body,
        grid=(GRID,),
        in_specs=[row(8), full((8, N_L)), full((1, N_L))],
        out_specs=[row(K), row(K)],
        out_shape=[jax.ShapeDtypeStruct((N_M, K), jnp.int32),
                   jax.ShapeDtypeStruct((N_M, K), jnp.float32)],
    )(qpad, pT, slab_row)

    sc_gather = pl.kernel(
        _sc_gather_body,
        out_type=jax.ShapeDtypeStruct((K * N_M, 2 * F1), jnp.float32),
        mesh=plsc.VectorSubcoreMesh(core_axis_name="c", subcore_axis_name="s"),
        scratch_types=[pltpu.VMEM((QPER_W,), jnp.int32),
                       pltpu.VMEM((QPER_W, 2 * F1), jnp.float32),
                       pltpu.SemaphoreType.DMA],
    )
    pf_pad = jnp.pad(prop_feats, ((0, 0), (0, 64)))
    gath = sc_gather(pf_pad, idxq.T.reshape(K * N_M))

    gblk = lambda k: pl.BlockSpec((TQ, 2 * F1), lambda i, k=k: (k * GRID + i, 0))
    y0, s0, ss0 = pl.pallas_call(
        _k2_body,
        grid=(GRID,),
        in_specs=[row(8), row(F2), gblk(0), gblk(1), gblk(2), row(K),
                  full((8, H)), full((F2, H)), full((F1, H)), full((1, H))],
        out_specs=[row(H), stat, stat],
        out_shape=[jax.ShapeDtypeStruct((N_M, H), jnp.float32),
                   statshape, statshape],
    )(qpad, orig_feats, gath, gath, gath, w3, w0c, w0f, w0i, b0[None, :])

    y1, s1, ss1 = pl.pallas_call(
        _k3_body,
        grid=(GRID,),
        in_specs=[row(H), full((GRID, 1, H)), full((GRID, 1, H)),
                  full((1, H)), full((1, H)), full((H, H)), full((1, H))],
        out_specs=[row(H), stat, stat],
        out_shape=[jax.ShapeDtypeStruct((N_M, H), jnp.float32),
                   statshape, statshape],
    )(y0, s0, ss0, g0[None, :], be0[None, :], W1, b1[None, :])

    out = pl.pallas_call(
        _k4_body,
        grid=(GRID,),
        in_specs=[row(H), full((GRID, 1, H)), full((GRID, 1, H)),
                  full((1, H)), full((1, H))],
        out_specs=row(H),
        out_shape=jax.ShapeDtypeStruct((N_M, H), jnp.float32),
    )(y1, s1, ss1, g1[None, :], be1[None, :])
    return out


# speedup vs baseline: 1.5594x; 1.0192x over previous
import jax
import jax.numpy as jnp
from jax import lax
from jax.experimental import pallas as pl
from jax.experimental.pallas import tpu as pltpu
from jax.experimental.pallas import tpu_sc as plsc

N_L = 4096
N_M = 16384
F1 = 64
F2 = 64
H = 128
TQ = 512
GRID = N_M // TQ
K = 3
EPS = 1e-5

NC = 2
NS = 16
NW = NC * NS
QPER_W = N_M // NW
CHUNK = 128
NCHUNK = QPER_W // CHUNK

_HIGHEST = jax.lax.Precision.HIGHEST
_DEF = jax.lax.Precision.DEFAULT

NSLAB = 32
SLABW = N_L // NSLAB


def _dot(a, b, precision=_DEF):
    return jax.lax.dot_general(a, b, (((1,), (0,)), ((), ())),
                               precision=precision,
                               preferred_element_type=jnp.float32)


def _k1_body(q_ref, pT_ref, slab_ref, idx_ref, w_ref):
    q = q_ref[...]
    pT = pT_ref[...]
    qn = jnp.sum(q * q, axis=1, keepdims=True)
    pn = jnp.sum(pT * pT, axis=0, keepdims=True)
    g = _dot(q, pT, _HIGHEST)
    d2 = jnp.maximum(qn + pn - 2.0 * g, 0.0)

    b = jax.lax.bitcast_convert_type(d2, jnp.int32)
    keys = jax.lax.bitcast_convert_type((b & ~(NSLAB - 1)) | slab_ref[...],
                                        jnp.float32)

    lane_iota = jax.lax.broadcasted_iota(jnp.int32, (TQ, SLABW), 1)
    idxs, ws = [], []
    wsum = jnp.zeros((TQ, 1), jnp.float32)
    for k in range(K):
        cmin = keys[:, :SLABW]
        for s in range(1, NSLAB):
            cmin = jnp.minimum(cmin, keys[:, s * SLABW:(s + 1) * SLABW])
        m = jnp.min(cmin, axis=1, keepdims=True)
        lane = jnp.min(jnp.where(cmin == m, lane_iota, SLABW),
                       axis=1, keepdims=True)
        mi = jax.lax.bitcast_convert_type(m, jnp.int32)
        slab = mi & (NSLAB - 1)
        d2t = jax.lax.bitcast_convert_type(mi & ~(NSLAB - 1), jnp.float32)
        idxs.append(slab * SLABW + lane)
        wk = 1.0 / jnp.maximum(d2t, 1e-12)
        ws.append(wk)
        wsum = wsum + wk
        if k < K - 1:
            keys = jnp.where(keys == m, jnp.inf, keys)
    inv = 1.0 / wsum
    idx_ref[...] = jnp.concatenate(idxs, axis=1)
    w_ref[...] = jnp.concatenate([w * inv for w in ws], axis=1)


def _sc_gather_body(pf_hbm, idx_hbm, out_hbm, idx_v, rows_v, sem):
    wid = lax.axis_index("s") * NC + lax.axis_index("c")
    qbase = wid * QPER_W
    for k in range(K):
        pltpu.sync_copy(idx_hbm.at[pl.ds(k * N_M + qbase, QPER_W)], idx_v)
        copies = []
        for c in range(NCHUNK):
            copies.append(pltpu.async_copy(
                pf_hbm.at[idx_v.at[pl.ds(c * CHUNK, CHUNK)]],
                rows_v.at[pl.ds(c * CHUNK, CHUNK)], sem))
        for cp in copies:
            cp.wait()
        pltpu.sync_copy(rows_v, out_hbm.at[pl.ds(k * N_M + qbase, QPER_W)])


def _k2_body(q_ref, of_ref, g0_ref, g1_ref, g2_ref, w_ref, w0c_ref,
             w0f_ref, w0i_ref, b0_ref, y0_ref, s_ref, ss_ref):
    w = w_ref[...]
    interp = (g0_ref[:, :F1] * w[:, 0:1] + g1_ref[:, :F1] * w[:, 1:2]
              + g2_ref[:, :F1] * w[:, 2:3])
    y0 = (_dot(q_ref[...], w0c_ref[...]) + _dot(of_ref[...], w0f_ref[...])
          + _dot(interp, w0i_ref[...]) + b0_ref[...])
    y0_ref[...] = y0
    s_ref[...] = jnp.sum(y0, axis=0, keepdims=True)[None]
    ss_ref[...] = jnp.sum(y0 * y0, axis=0, keepdims=True)[None]


def _bn_scale_shift(s_ref, ss_ref, g_ref, be_ref):
    mu = jnp.sum(s_ref[:, 0, :], axis=0, keepdims=True) * (1.0 / N_M)
    var = jnp.sum(ss_ref[:, 0, :], axis=0, keepdims=True) * (1.0 / N_M) - mu * mu
    scale = g_ref[...] / jnp.sqrt(var + EPS)
    shift = be_ref[...] - mu * scale
    return scale, shift


def _k3_body(y0_ref, s0_ref, ss0_ref, g0_ref, be0_ref, w1_ref, b1_ref,
             y1_ref, s_ref, ss_ref):
    scale, shift = _bn_scale_shift(s0_ref, ss0_ref, g0_ref, be0_ref)
    h = jnp.maximum(y0_ref[...] * scale + shift, 0.0)
    y1 = _dot(h, w1_ref[...]) + b1_ref[...]
    y1_ref[...] = y1
    s_ref[...] = jnp.sum(y1, axis=0, keepdims=True)[None]
    ss_ref[...] = jnp.sum(y1 * y1, axis=0, keepdims=True)[None]


def _k4_body(y1_ref, s1_ref, ss1_ref, g1_ref, be1_ref, o_ref):
    scale, shift = _bn_scale_shift(s1_ref, ss1_ref, g1_ref, be1_ref)
    o_ref[...] = jnp.maximum(y1_ref[...] * scale + shift, 0.0)


@jax.jit
def kernel(prop_coords, prop_feats, orig_coords, orig_feats,
           W0, b0, g0, be0, W1, b1, g1, be1):
    qpad = orig_coords
    pT = prop_coords.T
    w0c = W0[:3]
    w0f = W0[3:3 + F2]
    w0i = W0[3 + F2:]

    slab_row = (jnp.arange(N_L, dtype=jnp.int32) // SLABW)[None, :]
    full = lambda shp: pl.BlockSpec(shp, lambda i: (0,) * len(shp))
    row = lambda w: pl.BlockSpec((TQ, w), lambda i: (i, 0))
    stat = pl.BlockSpec((1, 1, H), lambda i: (i, 0, 0))
    statshape = jax.ShapeDtypeStruct((GRID, 1, H), jnp.float32)

    idxq, w3 = pl.pallas_call(
        _k1_body,
        grid=(GRID,),
        in_specs=[row(3), full((3, N_L)), full((1, N_L))],
        out_specs=[row(K), row(K)],
        out_shape=[jax.ShapeDtypeStruct((N_M, K), jnp.int32),
                   jax.ShapeDtypeStruct((N_M, K), jnp.float32)],
    )(qpad, pT, slab_row)

    sc_gather = pl.kernel(
        _sc_gather_body,
        out_type=jax.ShapeDtypeStruct((K * N_M, 2 * F1), jnp.float32),
        mesh=plsc.VectorSubcoreMesh(core_axis_name="c", subcore_axis_name="s"),
        scratch_types=[pltpu.VMEM((QPER_W,), jnp.int32),
                       pltpu.VMEM((QPER_W, 2 * F1), jnp.float32),
                       pltpu.SemaphoreType.DMA],
    )
    pf_pad = jnp.pad(prop_feats, ((0, 0), (0, 64)))
    gath = sc_gather(pf_pad, idxq.T.reshape(K * N_M))

    gblk = lambda k: pl.BlockSpec((TQ, 2 * F1), lambda i, k=k: (k * GRID + i, 0))
    y0, s0, ss0 = pl.pallas_call(
        _k2_body,
        grid=(GRID,),
        in_specs=[row(3), row(F2), gblk(0), gblk(1), gblk(2), row(K),
                  full((3, H)), full((F2, H)), full((F1, H)), full((1, H))],
        out_specs=[row(H), stat, stat],
        out_shape=[jax.ShapeDtypeStruct((N_M, H), jnp.float32),
                   statshape, statshape],
    )(qpad, orig_feats, gath, gath, gath, w3, w0c, w0f, w0i, b0[None, :])

    y1, s1, ss1 = pl.pallas_call(
        _k3_body,
        grid=(GRID,),
        in_specs=[row(H), full((GRID, 1, H)), full((GRID, 1, H)),
                  full((1, H)), full((1, H)), full((H, H)), full((1, H))],
        out_specs=[row(H), stat, stat],
        out_shape=[jax.ShapeDtypeStruct((N_M, H), jnp.float32),
                   statshape, statshape],
    )(y0, s0, ss0, g0[None, :], be0[None, :], W1, b1[None, :])

    out = pl.pallas_call(
        _k4_body,
        grid=(GRID,),
        in_specs=[row(H), full((GRID, 1, H)), full((GRID, 1, H)),
                  full((1, H)), full((1, H))],
        out_specs=row(H),
        out_shape=jax.ShapeDtypeStruct((N_M, H), jnp.float32),
    )(y1, s1, ss1, g1[None, :], be1[None, :])
    return out


# speedup vs baseline: 1.6399x; 1.0516x over previous
import jax
import jax.numpy as jnp
from jax import lax
from jax.experimental import pallas as pl
from jax.experimental.pallas import tpu as pltpu
from jax.experimental.pallas import tpu_sc as plsc

N_L = 4096
N_M = 16384
F1 = 64
F2 = 64
H = 128
TQ = 512
GRID = N_M // TQ
K = 3
EPS = 1e-5

NC = 2
NS = 16
NW = NC * NS
QPER_W = N_M // NW
CHUNK = 128
NCHUNK = QPER_W // CHUNK

_HIGHEST = jax.lax.Precision.HIGHEST
_DEF = jax.lax.Precision.DEFAULT

NSLAB = 32
SLABW = N_L // NSLAB


def _dot(a, b, precision=_DEF):
    return jax.lax.dot_general(a, b, (((1,), (0,)), ((), ())),
                               precision=precision,
                               preferred_element_type=jnp.float32)


def _k1_body(q_ref, pT_ref, slab_ref, idx_ref, w_ref):
    q = q_ref[...]
    pT = pT_ref[...]
    qn = jnp.sum(q * q, axis=1, keepdims=True)
    pn = jnp.sum(pT * pT, axis=0, keepdims=True)
    g = _dot(q, pT, _HIGHEST)
    d2 = jnp.maximum(qn + pn - 2.0 * g, 0.0)

    b = jax.lax.bitcast_convert_type(d2, jnp.int32)
    keys = jax.lax.bitcast_convert_type((b & ~(NSLAB - 1)) | slab_ref[...],
                                        jnp.float32)

    lane_iota = jax.lax.broadcasted_iota(jnp.int32, (TQ, SLABW), 1)
    idxs, ws = [], []
    wsum = jnp.zeros((TQ, 1), jnp.float32)
    for k in range(K):
        cmin = keys[:, :SLABW]
        for s in range(1, NSLAB):
            cmin = jnp.minimum(cmin, keys[:, s * SLABW:(s + 1) * SLABW])
        m = jnp.min(cmin, axis=1, keepdims=True)
        lane = jnp.min(jnp.where(cmin == m, lane_iota, SLABW),
                       axis=1, keepdims=True)
        mi = jax.lax.bitcast_convert_type(m, jnp.int32)
        slab = mi & (NSLAB - 1)
        d2t = jax.lax.bitcast_convert_type(mi & ~(NSLAB - 1), jnp.float32)
        idxs.append(slab * SLABW + lane)
        wk = 1.0 / jnp.maximum(d2t, 1e-12)
        ws.append(wk)
        wsum = wsum + wk
        if k < K - 1:
            keys = jnp.where(keys == m, jnp.inf, keys)
    inv = 1.0 / wsum
    idx_ref[...] = jnp.concatenate(idxs, axis=1)
    w_ref[...] = jnp.concatenate([w * inv for w in ws], axis=1)


def _sc_gather_body(pf_hbm, idx_hbm, out_hbm, idx_v, rows_v, sem):
    wid = lax.axis_index("s") * NC + lax.axis_index("c")
    qbase = wid * QPER_W
    for k in range(K):
        pltpu.sync_copy(idx_hbm.at[pl.ds(k * N_M + qbase, QPER_W)], idx_v)
        copies = []
        for c in range(NCHUNK):
            copies.append(pltpu.async_copy(
                pf_hbm.at[idx_v.at[pl.ds(c * CHUNK, CHUNK)]],
                rows_v.at[pl.ds(c * CHUNK, CHUNK)], sem))
        for cp in copies:
            cp.wait()
        pltpu.sync_copy(rows_v, out_hbm.at[pl.ds(k * N_M + qbase, QPER_W)])


def _bn_from_stats(st_ref, r, g_ref, be_ref):
    mu = st_ref[r:r + 1, :] * (1.0 / N_M)
    var = st_ref[r + 1:r + 2, :] * (1.0 / N_M) - mu * mu
    scale = g_ref[...] / jnp.sqrt(var + EPS)
    shift = be_ref[...] - mu * scale
    return scale, shift


def _mlp_body(q_ref, of_ref, g0_ref, g1_ref, g2_ref, w_ref, w0c_ref,
              w0f_ref, w0i_ref, b0_ref, g0p_ref, be0_ref, w1_ref, b1_ref,
              g1p_ref, be1_ref, o_ref, ya_ref, yb_ref, st_ref):
    p = pl.program_id(0)
    t = pl.program_id(1)
    rows = pl.ds(t * TQ, TQ)

    @pl.when(jnp.logical_and(p == 0, t == 0))
    def _init():
        st_ref[...] = jnp.zeros((8, H), jnp.float32)

    @pl.when(p == 0)
    def _phase0():
        w = w_ref[...]
        interp = (g0_ref[:, :F1] * w[:, 0:1] + g1_ref[:, :F1] * w[:, 1:2]
                  + g2_ref[:, :F1] * w[:, 2:3])
        y0 = (_dot(q_ref[...], w0c_ref[...]) + _dot(of_ref[...], w0f_ref[...])
              + _dot(interp, w0i_ref[...]) + b0_ref[...])
        ya_ref[rows, :] = y0
        st_ref[0:1, :] += jnp.sum(y0, axis=0, keepdims=True)
        st_ref[1:2, :] += jnp.sum(y0 * y0, axis=0, keepdims=True)

    @pl.when(p == 1)
    def _phase1():
        scale, shift = _bn_from_stats(st_ref, 0, g0p_ref, be0_ref)
        h = jnp.maximum(ya_ref[rows, :] * scale + shift, 0.0)
        y1 = _dot(h, w1_ref[...]) + b1_ref[...]
        yb_ref[rows, :] = y1
        st_ref[2:3, :] += jnp.sum(y1, axis=0, keepdims=True)
        st_ref[3:4, :] += jnp.sum(y1 * y1, axis=0, keepdims=True)

    @pl.when(p == 2)
    def _phase2():
        scale, shift = _bn_from_stats(st_ref, 2, g1p_ref, be1_ref)
        o_ref[...] = jnp.maximum(yb_ref[rows, :] * scale + shift, 0.0)


@jax.jit
def kernel(prop_coords, prop_feats, orig_coords, orig_feats,
           W0, b0, g0, be0, W1, b1, g1, be1):
    qpad = orig_coords
    pT = prop_coords.T
    w0c = W0[:3]
    w0f = W0[3:3 + F2]
    w0i = W0[3 + F2:]

    slab_row = (jnp.arange(N_L, dtype=jnp.int32) // SLABW)[None, :]
    full = lambda shp: pl.BlockSpec(shp, lambda i: (0,) * len(shp))
    row = lambda w: pl.BlockSpec((TQ, w), lambda i: (i, 0))

    idxq, w3 = pl.pallas_call(
        _k1_body,
        grid=(GRID,),
        in_specs=[row(3), full((3, N_L)), full((1, N_L))],
        out_specs=[row(K), row(K)],
        out_shape=[jax.ShapeDtypeStruct((N_M, K), jnp.int32),
                   jax.ShapeDtypeStruct((N_M, K), jnp.float32)],
    )(qpad, pT, slab_row)

    sc_gather = pl.kernel(
        _sc_gather_body,
        out_type=jax.ShapeDtypeStruct((K * N_M, 2 * F1), jnp.float32),
        mesh=plsc.VectorSubcoreMesh(core_axis_name="c", subcore_axis_name="s"),
        scratch_types=[pltpu.VMEM((QPER_W,), jnp.int32),
                       pltpu.VMEM((QPER_W, 2 * F1), jnp.float32),
                       pltpu.SemaphoreType.DMA],
    )
    pf_pad = jnp.pad(prop_feats, ((0, 0), (0, 64)))
    gath = sc_gather(pf_pad, idxq.T.reshape(K * N_M))

    rowp = lambda w: pl.BlockSpec((TQ, w),
                                  lambda p, i: (jnp.where(p == 0, i, 0), 0))
    fullp = lambda shp: pl.BlockSpec(shp, lambda p, i: (0,) * len(shp))
    gblkp = lambda k: pl.BlockSpec(
        (TQ, 2 * F1),
        lambda p, i, k=k: (k * GRID + jnp.where(p == 0, i, 0), 0))
    out = pl.pallas_call(
        _mlp_body,
        grid=(3, GRID),
        in_specs=[rowp(3), rowp(F2), gblkp(0), gblkp(1), gblkp(2), rowp(K),
                  fullp((3, H)), fullp((F2, H)), fullp((F1, H)),
                  fullp((1, H)), fullp((1, H)), fullp((1, H)),
                  fullp((H, H)), fullp((1, H)), fullp((1, H)),
                  fullp((1, H))],
        out_specs=pl.BlockSpec((TQ, H),
                               lambda p, i: (jnp.where(p == 2, i, 0), 0)),
        out_shape=jax.ShapeDtypeStruct((N_M, H), jnp.float32),
        scratch_shapes=[pltpu.VMEM((N_M, H), jnp.float32),
                        pltpu.VMEM((N_M, H), jnp.float32),
                        pltpu.VMEM((8, H), jnp.float32)],
    )(qpad, orig_feats, gath, gath, gath, w3, w0c, w0f, w0i, b0[None, :],
      g0[None, :], be0[None, :], W1, b1[None, :], g1[None, :], be1[None, :])
    return out


# speedup vs baseline: 1.6417x; 1.0011x over previous
import jax
import jax.numpy as jnp
import numpy as np
from jax import lax
from jax.experimental import pallas as pl
from jax.experimental.pallas import tpu as pltpu
from jax.experimental.pallas import tpu_sc as plsc

N_L = 4096
N_M = 16384
F1 = 64
F2 = 64
H = 128
TQ = 512
GRID = N_M // TQ
K = 3
EPS = 1e-5

NC = 2
NS = 16
NW = NC * NS
QPER_W = N_M // NW
CHUNK = 128
NCHUNK = QPER_W // CHUNK

_SLAB_ROW = (np.arange(N_L, dtype=np.int32) // (N_L // 32))[None, :]

_HIGHEST = jax.lax.Precision.HIGHEST
_DEF = jax.lax.Precision.DEFAULT

NSLAB = 32
SLABW = N_L // NSLAB


def _dot(a, b, precision=_DEF):
    return jax.lax.dot_general(a, b, (((1,), (0,)), ((), ())),
                               precision=precision,
                               preferred_element_type=jnp.float32)


def _k1_body(q_ref, pT_ref, slab_ref, idx_ref, w_ref):
    q = q_ref[...]
    pT = pT_ref[...]
    qn = jnp.sum(q * q, axis=1, keepdims=True)
    pn = jnp.sum(pT * pT, axis=0, keepdims=True)
    g = _dot(q, pT, _HIGHEST)
    d2 = jnp.maximum(qn + pn - 2.0 * g, 0.0)

    b = jax.lax.bitcast_convert_type(d2, jnp.int32)
    keys = jax.lax.bitcast_convert_type((b & ~(NSLAB - 1)) | slab_ref[...],
                                        jnp.float32)

    lane_iota = jax.lax.broadcasted_iota(jnp.int32, (TQ, SLABW), 1)
    idxs, ws = [], []
    wsum = jnp.zeros((TQ, 1), jnp.float32)
    for k in range(K):
        cmin = keys[:, :SLABW]
        for s in range(1, NSLAB):
            cmin = jnp.minimum(cmin, keys[:, s * SLABW:(s + 1) * SLABW])
        m = jnp.min(cmin, axis=1, keepdims=True)
        lane = jnp.min(jnp.where(cmin == m, lane_iota, SLABW),
                       axis=1, keepdims=True)
        mi = jax.lax.bitcast_convert_type(m, jnp.int32)
        slab = mi & (NSLAB - 1)
        d2t = jax.lax.bitcast_convert_type(mi & ~(NSLAB - 1), jnp.float32)
        idxs.append(slab * SLABW + lane)
        wk = 1.0 / jnp.maximum(d2t, 1e-12)
        ws.append(wk)
        wsum = wsum + wk
        if k < K - 1:
            keys = jnp.where(keys == m, jnp.inf, keys)
    inv = 1.0 / wsum
    idx_ref[...] = jnp.concatenate(idxs, axis=1)
    w_ref[...] = jnp.concatenate([w * inv for w in ws], axis=1)


def _sc_gather_body(pf_hbm, idx_hbm, out_hbm, idx_v, rows_v, sem):
    wid = lax.axis_index("s") * NC + lax.axis_index("c")
    qbase = wid * QPER_W
    for k in range(K):
        pltpu.sync_copy(idx_hbm.at[pl.ds(k * N_M + qbase, QPER_W)], idx_v)
        copies = []
        for c in range(NCHUNK):
            copies.append(pltpu.async_copy(
                pf_hbm.at[idx_v.at[pl.ds(c * CHUNK, CHUNK)]],
                rows_v.at[pl.ds(c * CHUNK, CHUNK)], sem))
        for cp in copies:
            cp.wait()
        pltpu.sync_copy(rows_v, out_hbm.at[pl.ds(k * N_M + qbase, QPER_W)])


def _bn_from_stats(st_ref, r, g_ref, be_ref):
    mu = st_ref[r:r + 1, :] * (1.0 / N_M)
    var = st_ref[r + 1:r + 2, :] * (1.0 / N_M) - mu * mu
    scale = g_ref[...] / jnp.sqrt(var + EPS)
    shift = be_ref[...] - mu * scale
    return scale, shift


def _mlp_body(q_ref, of_ref, g0_ref, g1_ref, g2_ref, w_ref, w0c_ref,
              w0f_ref, w0i_ref, b0_ref, g0p_ref, be0_ref, w1_ref, b1_ref,
              g1p_ref, be1_ref, o_ref, ya_ref, yb_ref, st_ref):
    p = pl.program_id(0)
    t = pl.program_id(1)
    rows = pl.ds(t * TQ, TQ)

    @pl.when(jnp.logical_and(p == 0, t == 0))
    def _init():
        st_ref[...] = jnp.zeros((8, H), jnp.float32)

    @pl.when(p == 0)
    def _phase0():
        w = w_ref[...]
        interp = (g0_ref[:, :F1] * w[:, 0:1] + g1_ref[:, :F1] * w[:, 1:2]
                  + g2_ref[:, :F1] * w[:, 2:3])
        y0 = (_dot(q_ref[...], w0c_ref[...]) + _dot(of_ref[...], w0f_ref[...])
              + _dot(interp, w0i_ref[...]) + b0_ref[...])
        ya_ref[rows, :] = y0
        st_ref[0:1, :] += jnp.sum(y0, axis=0, keepdims=True)
        st_ref[1:2, :] += jnp.sum(y0 * y0, axis=0, keepdims=True)

    @pl.when(p == 1)
    def _phase1():
        scale, shift = _bn_from_stats(st_ref, 0, g0p_ref, be0_ref)
        h = jnp.maximum(ya_ref[rows, :] * scale + shift, 0.0)
        y1 = _dot(h, w1_ref[...]) + b1_ref[...]
        yb_ref[rows, :] = y1
        st_ref[2:3, :] += jnp.sum(y1, axis=0, keepdims=True)
        st_ref[3:4, :] += jnp.sum(y1 * y1, axis=0, keepdims=True)

    @pl.when(p == 2)
    def _phase2():
        scale, shift = _bn_from_stats(st_ref, 2, g1p_ref, be1_ref)
        o_ref[...] = jnp.maximum(yb_ref[rows, :] * scale + shift, 0.0)


@jax.jit
def kernel(prop_coords, prop_feats, orig_coords, orig_feats,
           W0, b0, g0, be0, W1, b1, g1, be1):
    qpad = orig_coords
    pT = prop_coords.T
    w0c = W0[:3]
    w0f = W0[3:3 + F2]
    w0i = W0[3 + F2:]

    slab_row = _SLAB_ROW
    full = lambda shp: pl.BlockSpec(shp, lambda i: (0,) * len(shp))
    row = lambda w: pl.BlockSpec((TQ, w), lambda i: (i, 0))

    idxq, w3 = pl.pallas_call(
        _k1_body,
        grid=(GRID,),
        in_specs=[row(3), full((3, N_L)), full((1, N_L))],
        out_specs=[row(K), row(K)],
        out_shape=[jax.ShapeDtypeStruct((N_M, K), jnp.int32),
                   jax.ShapeDtypeStruct((N_M, K), jnp.float32)],
    )(qpad, pT, slab_row)

    sc_gather = pl.kernel(
        _sc_gather_body,
        out_type=jax.ShapeDtypeStruct((K * N_M, 2 * F1), jnp.float32),
        mesh=plsc.VectorSubcoreMesh(core_axis_name="c", subcore_axis_name="s"),
        scratch_types=[pltpu.VMEM((QPER_W,), jnp.int32),
                       pltpu.VMEM((QPER_W, 2 * F1), jnp.float32),
                       pltpu.SemaphoreType.DMA],
    )
    pf_pad = jnp.pad(prop_feats, ((0, 0), (0, 64)))
    gath = sc_gather(pf_pad, idxq.T.reshape(K * N_M))

    rowp = lambda w: pl.BlockSpec((TQ, w),
                                  lambda p, i: (jnp.where(p == 0, i, 0), 0))
    fullp = lambda shp: pl.BlockSpec(shp, lambda p, i: (0,) * len(shp))
    gblkp = lambda k: pl.BlockSpec(
        (TQ, 2 * F1),
        lambda p, i, k=k: (k * GRID + jnp.where(p == 0, i, 0), 0))
    out = pl.pallas_call(
        _mlp_body,
        grid=(3, GRID),
        in_specs=[rowp(3), rowp(F2), gblkp(0), gblkp(1), gblkp(2), rowp(K),
                  fullp((3, H)), fullp((F2, H)), fullp((F1, H)),
                  fullp((1, H)), fullp((1, H)), fullp((1, H)),
                  fullp((H, H)), fullp((1, H)), fullp((1, H)),
                  fullp((1, H))],
        out_specs=pl.BlockSpec((TQ, H),
                               lambda p, i: (jnp.where(p == 2, i, 0), 0)),
        out_shape=jax.ShapeDtypeStruct((N_M, H), jnp.float32),
        scratch_shapes=[pltpu.VMEM((N_M, H), jnp.float32),
                        pltpu.VMEM((N_M, H), jnp.float32),
                        pltpu.VMEM((8, H), jnp.float32)],
    )(qpad, orig_feats, gath, gath, gath, w3, w0c, w0f, w0i, b0[None, :],
      g0[None, :], be0[None, :], W1, b1[None, :], g1[None, :], be1[None, :])
    return out
